# baseline (speedup 1.0000x reference)
import functools
import math

import jax
import jax.numpy as jnp
from jax import lax
from jax.experimental import pallas as pl
from jax.experimental.pallas import tpu as pltpu
from jax.experimental.pallas import tpu_sc as plsc

D_MODEL = 64
SCALE = math.sqrt(D_MODEL)

NUM_CORES = 2
NUM_SUBCORES = 16
NUM_WORKERS = NUM_CORES * NUM_SUBCORES

CHUNK = 512
GATHER_ROWS = 128
SUBGATHERS = CHUNK // GATHER_ROWS
LANES = 16


def _build_kernel(n_rows):
    assert n_rows % (NUM_WORKERS * CHUNK) == 0
    rows_per_worker = n_rows // NUM_WORKERS
    n_chunks = rows_per_worker // CHUNK
    assert n_chunks % 2 == 0
    mesh = plsc.VectorSubcoreMesh(core_axis_name="c", subcore_axis_name="s")

    @functools.partial(
        pl.kernel,
        mesh=mesh,
        compiler_params=pltpu.CompilerParams(use_tc_tiling_on_sc=False),
        out_type=jax.ShapeDtypeStruct((n_rows, D_MODEL), jnp.float32),
        scratch_types=[
            pltpu.VMEM((rows_per_worker,), jnp.int32),
            pltpu.VMEM((CHUNK, D_MODEL), jnp.float32),
            pltpu.VMEM((CHUNK, D_MODEL), jnp.float32),
            pltpu.SemaphoreType.DMA,
            pltpu.SemaphoreType.DMA,
        ],
    )
    def emb_kernel(idx_hbm, lut_hbm, out_hbm, idx_v, buf0, buf1, sem0, sem1):
        wid = lax.axis_index("s") * NUM_CORES + lax.axis_index("c")
        base = wid * rows_per_worker
        pltpu.sync_copy(idx_hbm.at[pl.ds(base, rows_per_worker)], idx_v)

        def start_gather(chunk, buf, sem):
            for j in range(SUBGATHERS):
                idx_slice = idx_v.at[pl.ds(chunk * CHUNK + j * GATHER_ROWS,
                                           GATHER_ROWS)]
                pltpu.async_copy(
                    lut_hbm.at[idx_slice],
                    buf.at[pl.ds(j * GATHER_ROWS, GATHER_ROWS)],
                    sem,
                )

        def wait_gather(buf, sem):
            pltpu.make_async_copy(lut_hbm.at[pl.ds(0, CHUNK)], buf, sem).wait()

        def scale(buf):
            def row_body(i, carry):
                for j in range(D_MODEL // LANES):
                    sl = (i, pl.ds(j * LANES, LANES))
                    buf[sl] = buf[sl] * SCALE
                return carry
            lax.fori_loop(0, CHUNK, row_body, 0)

        def finish_chunk(chunk, buf, sem):
            wait_gather(buf, sem)
            scale(buf)
            pltpu.sync_copy(buf, out_hbm.at[pl.ds(base + chunk * CHUNK, CHUNK)])

        start_gather(0, buf0, sem0)

        def pair_body(p, carry):
            g = p * 2
            start_gather(g + 1, buf1, sem1)
            finish_chunk(g, buf0, sem0)

            @pl.when(g + 2 < n_chunks)
            def _():
                start_gather(g + 2, buf0, sem0)

            finish_chunk(g + 1, buf1, sem1)
            return carry

        lax.fori_loop(0, n_chunks // 2, pair_body, 0)

    return emb_kernel


def kernel(x, lut):
    n_rows = x.shape[0] * x.shape[1]
    flat_idx = x.reshape(n_rows)
    out = _build_kernel(n_rows)(flat_idx, lut)
    return out.reshape(x.shape[0], x.shape[1], D_MODEL)
